# Initial kernel scaffold; baseline (speedup 1.0000x reference)
import jax
import jax.numpy as jnp
from jax.experimental import pallas as pl


def kernel(x, p, c, x_h, freqs_q, freqs_v, Wq1, bq1, Wq2, bq2, Wv1e, bv1e, Wv2e, bv2e, Wq, bq, Wk, bk, Wv, bv, Wc1, bc1, gc, bcl, Wc2, bc2, Wf1, bf1, gf, bfl, Wf2, bf2, Wm1, bm1, gm, bml, Wm2, bm2, Wo, bo):
    raise NotImplementedError("write your pallas kernel here")


# speedup vs baseline: 4.3914x; 4.3914x over previous
import functools

import jax
import jax.numpy as jnp
from jax import lax
from jax.experimental import pallas as pl
from jax.experimental.pallas import tpu as pltpu
from jax.experimental.pallas import tpu_sc as plsc

TOP_K = 9
FREQ_Q = 0.1
FREQ_V = 1.0
TWO_PI = 2.0 * 3.141592653589793
IDX_PAD = 16

QB = 512
NB = 128


def _topk_body(x_ref, pT_ref, out_ref):
    qb = x_ref.shape[1]
    L = pT_ref.shape[2]
    xb = x_ref[0]
    acc = None
    for d in range(x_ref.shape[2]):
        xd = xb[:, d:d + 1]
        pd = pT_ref[0, d:d + 1, :]
        diff = xd - pd
        acc = diff * diff if acc is None else acc + diff * diff
    iota = lax.broadcasted_iota(jnp.int32, (qb, L), 1)
    big = jnp.float32(3.0e38)
    for j in range(TOP_K):
        m = jnp.min(acc, axis=1, keepdims=True)
        idxj = jnp.min(jnp.where(acc == m, iota, L), axis=1, keepdims=True)
        out_ref[0, :, j:j + 1] = idxj
        acc = jnp.where(iota == idxj, big, acc)
    out_ref[0, :, TOP_K:] = jnp.zeros((qb, IDX_PAD - TOP_K), jnp.int32)


def _ln(x, g, b, eps=1e-6):
    m = jnp.mean(x, axis=1, keepdims=True)
    v = jnp.mean((x - m) * (x - m), axis=1, keepdims=True)
    return (x - m) / jnp.sqrt(v + eps) * g + b


def _mm(a, w):
    return jax.lax.dot_general(a, w, (((1,), (0,)), ((), ())),
                               preferred_element_type=jnp.float32)


def _emb(inv, f_ref, mult, W1_ref, b1_ref, W2_ref, b2_ref):
    proj = None
    for d in range(3):
        t = inv[:, d:d + 1] * f_ref[d:d + 1, :]
        proj = t if proj is None else proj + t
    proj = proj * (TWO_PI * mult)
    h = jnp.concatenate([jnp.sin(proj), jnp.cos(proj)], axis=1)
    h = jax.nn.gelu(_mm(h, W1_ref[...]) + b1_ref[...])
    return _mm(h, W2_ref[...]) + b2_ref[...]


def _dense_body(g_ref, x_ref, xh_ref, fq_ref, fv_ref,
                Wq1_ref, bq1_ref, Wq2_ref, bq2_ref,
                Wv1e_ref, bv1e_ref, Wv2e_ref, bv2e_ref,
                Wq_ref, bq_ref, Wk_ref, bk_ref, Wv_ref, bv_ref,
                Wc1_ref, bc1_ref, gc_ref, bcl_ref, Wc2_ref, bc2_ref,
                Wf1_ref, bf1_ref, gf_ref, bfl_ref, Wf2_ref, bf2_ref,
                Wm1_ref, bm1_ref, gm_ref, bml_ref, Wm2_ref, bm2_ref,
                Wo_ref, bo_ref, out_ref):
    K, nb, _ = g_ref.shape
    H = Wq1_ref.shape[0]
    HH = Wq_ref.shape[1]
    NH = HH // H
    R = K * nb
    scale = 1.0 / (H ** 0.5)

    g = g_ref[...].reshape(R, g_ref.shape[2])
    cg = g[:, :H]
    pg = g[:, H:H + 3]
    xq = x_ref[...]
    xt = jnp.concatenate([xq] * K, axis=0)
    inv = xt - pg

    embq = _emb(inv, fq_ref, FREQ_Q, Wq1_ref, bq1_ref, Wq2_ref, bq2_ref)
    q = _mm(embq, Wq_ref[...]) + bq_ref[...]
    k = _mm(cg, Wk_ref[...]) + bk_ref[...]
    v = _mm(cg, Wv_ref[...]) + bv_ref[...]

    embv = _emb(inv, fv_ref, FREQ_V, Wv1e_ref, bv1e_ref, Wv2e_ref, bv2e_ref)

    xh = xh_ref[...]
    hc = jax.nn.gelu(_mm(xh, Wc1_ref[...]) + bc1_ref[...])
    hc = _ln(hc, gc_ref[...], bcl_ref[...])
    gb = _mm(hc, Wc2_ref[...]) + bc2_ref[...]
    gamma = gb[:, :H]
    beta = gb[:, H:]
    gamma_t = jnp.concatenate([gamma] * K, axis=0)
    beta_t = jnp.concatenate([beta] * K, axis=0)
    embv = embv * (1.0 + gamma_t) + beta_t

    hf = jax.nn.gelu(_mm(embv, Wf1_ref[...]) + bf1_ref[...])
    hf = _ln(hf, gf_ref[...], bfl_ref[...])
    vgb = _mm(hf, Wf2_ref[...]) + bf2_ref[...]
    vg = vgb[:, :HH]
    vb = vgb[:, HH:]
    v = v * (1.0 + vg) + vb

    prod = q * k
    y_parts = []
    for h in range(NH):
        sl = slice(h * H, (h + 1) * H)
        vh = jax.nn.gelu(_mm(v[:, sl], Wm1_ref[...]) + bm1_ref[...])
        vh = _ln(vh, gm_ref[...], bml_ref[...])
        vh = _mm(vh, Wm2_ref[...]) + bm2_ref[...]
        ah = jnp.sum(prod[:, sl], axis=1, keepdims=True) * scale
        aks = [ah[kk * nb:(kk + 1) * nb] for kk in range(K)]
        m = aks[0]
        for kk in range(1, K):
            m = jnp.maximum(m, aks[kk])
        es = [jnp.exp(a - m) for a in aks]
        s = es[0]
        for kk in range(1, K):
            s = s + es[kk]
        yh = None
        for kk in range(K):
            w_kk = es[kk] / s
            contrib = w_kk * vh[kk * nb:(kk + 1) * nb, :]
            yh = contrib if yh is None else yh + contrib
        y_parts.append(yh)
    y = jnp.concatenate(y_parts, axis=1)
    out_ref[...] = _mm(y, Wo_ref[...]) + bo_ref[...]


def _sc_gather(table, idx_flat):
    M = idx_flat.shape[0]
    W = table.shape[1]
    NC, NS = 2, 16
    NW = NC * NS
    per_w = M // NW
    CH = 128
    n_ch = per_w // CH
    mesh = plsc.VectorSubcoreMesh(core_axis_name="c", subcore_axis_name="s")

    @functools.partial(
        pl.kernel, mesh=mesh,
        out_type=jax.ShapeDtypeStruct((M, W), jnp.float32),
        scratch_types=[
            pltpu.VMEM((CH,), jnp.int32),
            pltpu.VMEM((CH, W), jnp.float32),
            pltpu.SemaphoreType.DMA,
        ],
    )
    def gk(t_hbm, i_hbm, o_hbm, idx_v, rows_v, sem):
        wid = lax.axis_index("s") * NC + lax.axis_index("c")
        base_w = wid * per_w
        for j in range(n_ch):
            b = base_w + j * CH
            pltpu.sync_copy(i_hbm.at[pl.ds(b, CH)], idx_v)
            pltpu.async_copy(t_hbm.at[idx_v], rows_v, sem).wait()
            pltpu.sync_copy(rows_v, o_hbm.at[pl.ds(b, CH)])

    return gk(table, idx_flat)


def kernel(x, p, c, x_h, freqs_q, freqs_v, Wq1, bq1, Wq2, bq2, Wv1e, bv1e, Wv2e, bv2e, Wq, bq, Wk, bk, Wv, bv, Wc1, bc1, gc, bcl, Wc2, bc2, Wf1, bf1, gf, bfl, Wf2, bf2, Wm1, bm1, gm, bml, Wm2, bm2, Wo, bo):
    B, N, D = x.shape
    L = p.shape[1]
    H = c.shape[2]
    HH = Wq.shape[1]
    BN = B * N

    pT = jnp.swapaxes(p, 1, 2)
    idx16 = pl.pallas_call(
        _topk_body,
        grid=(B, N // QB),
        in_specs=[
            pl.BlockSpec((1, QB, D), lambda b, i: (b, i, 0)),
            pl.BlockSpec((1, D, L), lambda b, i: (b, 0, 0)),
        ],
        out_specs=pl.BlockSpec((1, QB, IDX_PAD), lambda b, i: (b, i, 0)),
        out_shape=jax.ShapeDtypeStruct((B, N, IDX_PAD), jnp.int32),
    )(x, pT)
    idx9 = idx16[:, :, :TOP_K]
    offs = (jnp.arange(B, dtype=jnp.int32) * L)[:, None, None]
    idx_flat = (idx9 + offs).reshape(BN, TOP_K).T.reshape(-1)

    table = jnp.concatenate(
        [c.reshape(B * L, H), p.reshape(B * L, D),
         jnp.zeros((B * L, 128 - H - D), jnp.float32)], axis=1)
    G = _sc_gather(table, idx_flat)
    G3 = G.reshape(TOP_K, BN, 128)

    x_flat = x.reshape(BN, D)
    xh_flat = x_h.reshape(BN, H)
    row = lambda a: a.reshape(1, -1)
    full = lambda arr: pl.BlockSpec(arr.shape, lambda i: (0,) * arr.ndim)
    weights = [freqs_q, freqs_v,
               Wq1, row(bq1), Wq2, row(bq2),
               Wv1e, row(bv1e), Wv2e, row(bv2e),
               Wq, row(bq), Wk, row(bk), Wv, row(bv),
               Wc1, row(bc1), row(gc), row(bcl), Wc2, row(bc2),
               Wf1, row(bf1), row(gf), row(bfl), Wf2, row(bf2),
               Wm1, row(bm1), row(gm), row(bml), Wm2, row(bm2),
               Wo, row(bo)]
    out = pl.pallas_call(
        _dense_body,
        grid=(BN // NB,),
        in_specs=[
            pl.BlockSpec((TOP_K, NB, 128), lambda i: (0, i, 0)),
            pl.BlockSpec((NB, D), lambda i: (i, 0)),
            pl.BlockSpec((NB, H), lambda i: (i, 0)),
        ] + [full(a) for a in weights],
        out_specs=pl.BlockSpec((NB, H), lambda i: (i, 0)),
        out_shape=jax.ShapeDtypeStruct((BN, H), jnp.float32),
    )(G3, x_flat, xh_flat, *weights)
    return out.reshape(B, N, H)


# speedup vs baseline: 4.9206x; 1.1205x over previous
import functools

import jax
import jax.numpy as jnp
from jax import lax
from jax.experimental import pallas as pl
from jax.experimental.pallas import tpu as pltpu
from jax.experimental.pallas import tpu_sc as plsc

TOP_K = 9
FREQ_Q = 0.1
FREQ_V = 1.0
TWO_PI = 2.0 * 3.141592653589793
IDX_PAD = 16

QB = 512
NB = 128


def _topk_body(x_ref, pT_ref, out_ref):
    qb = x_ref.shape[1]
    L = pT_ref.shape[2]
    xb = x_ref[0]
    acc = None
    for d in range(x_ref.shape[2]):
        xd = xb[:, d:d + 1]
        pd = pT_ref[0, d:d + 1, :]
        diff = xd - pd
        acc = diff * diff if acc is None else acc + diff * diff
    iota = lax.broadcasted_iota(jnp.int32, (qb, L), 1)
    big = jnp.float32(3.0e38)
    for j in range(TOP_K):
        m = jnp.min(acc, axis=1, keepdims=True)
        idxj = jnp.min(jnp.where(acc == m, iota, L), axis=1, keepdims=True)
        out_ref[0, :, j:j + 1] = idxj
        acc = jnp.where(iota == idxj, big, acc)
    out_ref[0, :, TOP_K:] = jnp.zeros((qb, IDX_PAD - TOP_K), jnp.int32)


def _ln(x, g, b, eps=1e-6):
    m = jnp.mean(x, axis=1, keepdims=True)
    xc = x - m
    v = jnp.mean(xc * xc, axis=1, keepdims=True)
    return xc * lax.rsqrt(v + eps) * g + b


def _mm(a, w):
    return jax.lax.dot_general(a, w, (((1,), (0,)), ((), ())),
                               preferred_element_type=jnp.float32)


def _emb(inv, f_ref, ph_ref, W1_ref, b1_ref, W2_ref, b2_ref):
    proj = None
    for d in range(3):
        t = inv[:, d:d + 1] * f_ref[d:d + 1, :]
        proj = t if proj is None else proj + t
    h = jnp.sin(proj + ph_ref[...])
    h = jax.nn.gelu(_mm(h, W1_ref[...]) + b1_ref[...])
    return _mm(h, W2_ref[...]) + b2_ref[...]


def _dense_body(g_ref, x_ref, xh_ref, fq_ref, fv_ref, ph_ref,
                Wq1_ref, bq1_ref, Wq2_ref, bq2_ref,
                Wv1e_ref, bv1e_ref, Wv2e_ref, bv2e_ref,
                Wq_ref, bq_ref, Wk_ref, bk_ref, Wv_ref, bv_ref,
                Wc1_ref, bc1_ref, gc_ref, bcl_ref, Wc2_ref, bc2_ref,
                Wf1_ref, bf1_ref, gf_ref, bfl_ref, Wf2_ref, bf2_ref,
                Wm1_ref, bm1_ref, gm_ref, bml_ref, Wm2_ref, bm2_ref,
                Wo_ref, bo_ref, out_ref):
    K, nb, _ = g_ref.shape
    H = Wq1_ref.shape[0]
    HH = Wq_ref.shape[1]
    NH = HH // H
    R = K * nb
    scale = 1.0 / (H ** 0.5)

    g = g_ref[...].reshape(R, g_ref.shape[2])
    cg = g[:, :H]
    pg = g[:, H:H + 3]
    xq = x_ref[...]
    xt = jnp.concatenate([xq] * K, axis=0)
    inv = xt - pg

    embq = _emb(inv, fq_ref, ph_ref, Wq1_ref, bq1_ref, Wq2_ref, bq2_ref)
    q = _mm(embq, Wq_ref[...]) + bq_ref[...]
    k = _mm(cg, Wk_ref[...]) + bk_ref[...]
    v = _mm(cg, Wv_ref[...]) + bv_ref[...]

    embv = _emb(inv, fv_ref, ph_ref, Wv1e_ref, bv1e_ref, Wv2e_ref, bv2e_ref)

    xh = xh_ref[...]
    hc = jax.nn.gelu(_mm(xh, Wc1_ref[...]) + bc1_ref[...])
    hc = _ln(hc, gc_ref[...], bcl_ref[...])
    gb = _mm(hc, Wc2_ref[...]) + bc2_ref[...]
    gamma = gb[:, :H]
    beta = gb[:, H:]
    gamma_t = jnp.concatenate([gamma] * K, axis=0)
    beta_t = jnp.concatenate([beta] * K, axis=0)
    embv = embv * (1.0 + gamma_t) + beta_t

    hf = jax.nn.gelu(_mm(embv, Wf1_ref[...]) + bf1_ref[...])
    hf = _ln(hf, gf_ref[...], bfl_ref[...])
    vgb = _mm(hf, Wf2_ref[...]) + bf2_ref[...]
    vg = vgb[:, :HH]
    vb = vgb[:, HH:]
    v = v * (1.0 + vg) + vb

    prod = q * k
    out = None
    for h in range(NH):
        sl = slice(h * H, (h + 1) * H)
        vh = jax.nn.gelu(_mm(v[:, sl], Wm1_ref[...]) + bm1_ref[...])
        vh = _ln(vh, gm_ref[...], bml_ref[...])
        vh = _mm(vh, Wm2_ref[...]) + bm2_ref[...]
        ah = jnp.sum(prod[:, sl], axis=1, keepdims=True) * scale
        aks = [ah[kk * nb:(kk + 1) * nb] for kk in range(K)]
        m = aks[0]
        for kk in range(1, K):
            m = jnp.maximum(m, aks[kk])
        es = [jnp.exp(a - m) for a in aks]
        s = es[0]
        for kk in range(1, K):
            s = s + es[kk]
        rin = 1.0 / s
        yh = None
        for kk in range(K):
            w_kk = es[kk] * rin
            contrib = w_kk * vh[kk * nb:(kk + 1) * nb, :]
            yh = contrib if yh is None else yh + contrib
        part = _mm(yh, Wo_ref[h * H:(h + 1) * H, :])
        out = part if out is None else out + part
    out_ref[...] = out + bo_ref[...]


def _sc_gather(table, idx_flat):
    M = idx_flat.shape[0]
    W = table.shape[1]
    NC, NS = 2, 16
    NW = NC * NS
    per_w = M // NW
    CH = 128
    n_ch = per_w // CH
    mesh = plsc.VectorSubcoreMesh(core_axis_name="c", subcore_axis_name="s")

    @functools.partial(
        pl.kernel, mesh=mesh,
        out_type=jax.ShapeDtypeStruct((M, W), jnp.float32),
        scratch_types=[
            pltpu.VMEM((CH,), jnp.int32),
            pltpu.VMEM((CH, W), jnp.float32),
            pltpu.SemaphoreType.DMA,
        ],
    )
    def gk(t_hbm, i_hbm, o_hbm, idx_v, rows_v, sem):
        wid = lax.axis_index("s") * NC + lax.axis_index("c")
        base_w = wid * per_w
        for j in range(n_ch):
            b = base_w + j * CH
            pltpu.sync_copy(i_hbm.at[pl.ds(b, CH)], idx_v)
            pltpu.async_copy(t_hbm.at[idx_v], rows_v, sem).wait()
            pltpu.sync_copy(rows_v, o_hbm.at[pl.ds(b, CH)])

    return gk(table, idx_flat)


def kernel(x, p, c, x_h, freqs_q, freqs_v, Wq1, bq1, Wq2, bq2, Wv1e, bv1e, Wv2e, bv2e, Wq, bq, Wk, bk, Wv, bv, Wc1, bc1, gc, bcl, Wc2, bc2, Wf1, bf1, gf, bfl, Wf2, bf2, Wm1, bm1, gm, bml, Wm2, bm2, Wo, bo):
    B, N, D = x.shape
    L = p.shape[1]
    H = c.shape[2]
    HH = Wq.shape[1]
    BN = B * N

    pT = jnp.swapaxes(p, 1, 2)
    idx16 = pl.pallas_call(
        _topk_body,
        grid=(B, N // QB),
        in_specs=[
            pl.BlockSpec((1, QB, D), lambda b, i: (b, i, 0)),
            pl.BlockSpec((1, D, L), lambda b, i: (b, 0, 0)),
        ],
        out_specs=pl.BlockSpec((1, QB, IDX_PAD), lambda b, i: (b, i, 0)),
        out_shape=jax.ShapeDtypeStruct((B, N, IDX_PAD), jnp.int32),
    )(x, pT)
    idx9 = idx16[:, :, :TOP_K]
    offs = (jnp.arange(B, dtype=jnp.int32) * L)[:, None, None]
    idx_flat = (idx9 + offs).reshape(BN, TOP_K).T.reshape(-1)

    table = jnp.concatenate(
        [c.reshape(B * L, H), p.reshape(B * L, D),
         jnp.zeros((B * L, 128 - H - D), jnp.float32)], axis=1)
    G = _sc_gather(table, idx_flat)
    G3 = G.reshape(TOP_K, BN, 128)

    x_flat = x.reshape(BN, D)
    xh_flat = x_h.reshape(BN, H)
    row = lambda a: a.reshape(1, -1)
    full = lambda arr: pl.BlockSpec(arr.shape, lambda i: (0,) * arr.ndim)
    fq2 = (TWO_PI * FREQ_Q) * jnp.concatenate([freqs_q, freqs_q], axis=1)
    fv2 = (TWO_PI * FREQ_V) * jnp.concatenate([freqs_v, freqs_v], axis=1)
    nhalf = freqs_q.shape[1]
    phase = jnp.concatenate([jnp.zeros((1, nhalf), jnp.float32),
                             jnp.full((1, nhalf), 0.5 * 3.141592653589793,
                                      jnp.float32)], axis=1)
    weights = [fq2, fv2, phase,
               Wq1, row(bq1), Wq2, row(bq2),
               Wv1e, row(bv1e), Wv2e, row(bv2e),
               Wq, row(bq), Wk, row(bk), Wv, row(bv),
               Wc1, row(bc1), row(gc), row(bcl), Wc2, row(bc2),
               Wf1, row(bf1), row(gf), row(bfl), Wf2, row(bf2),
               Wm1, row(bm1), row(gm), row(bml), Wm2, row(bm2),
               Wo, row(bo)]
    out = pl.pallas_call(
        _dense_body,
        grid=(BN // NB,),
        in_specs=[
            pl.BlockSpec((TOP_K, NB, 128), lambda i: (0, i, 0)),
            pl.BlockSpec((NB, D), lambda i: (i, 0)),
            pl.BlockSpec((NB, H), lambda i: (i, 0)),
        ] + [full(a) for a in weights],
        out_specs=pl.BlockSpec((NB, H), lambda i: (i, 0)),
        out_shape=jax.ShapeDtypeStruct((BN, H), jnp.float32),
    )(G3, x_flat, xh_flat, *weights)
    return out.reshape(B, N, H)


# speedup vs baseline: 6.9978x; 1.4221x over previous
import functools

import jax
import jax.numpy as jnp
from jax import lax
from jax.experimental import pallas as pl
from jax.experimental.pallas import tpu as pltpu
from jax.experimental.pallas import tpu_sc as plsc

TOP_K = 9
FREQ_Q = 0.1
FREQ_V = 1.0
PI = 3.141592653589793
TWO_PI = 2.0 * PI
IDX_PAD = 16

QB = 512
NB = 128


def _topk_body(x_ref, pT_ref, out_ref):
    qb = x_ref.shape[1]
    L = pT_ref.shape[2]
    xb = x_ref[0]
    acc = None
    for d in range(x_ref.shape[2]):
        xd = xb[:, d:d + 1]
        pd = pT_ref[0, d:d + 1, :]
        diff = xd - pd
        acc = diff * diff if acc is None else acc + diff * diff
    acc = jnp.sqrt(acc)
    iota = lax.broadcasted_iota(jnp.int32, (qb, L), 1)
    big = jnp.float32(3.0e38)
    for j in range(TOP_K):
        m = jnp.min(acc, axis=1, keepdims=True)
        idxj = jnp.min(jnp.where(acc == m, iota, L), axis=1, keepdims=True)
        out_ref[0, :, j:j + 1] = idxj
        acc = jnp.where(iota == idxj, big, acc)
    out_ref[0, :, TOP_K:] = jnp.zeros((qb, IDX_PAD - TOP_K), jnp.int32)


def _mm(a, w):
    return jax.lax.dot_general(a, w, (((1,), (0,)), ((), ())),
                               preferred_element_type=jnp.float32)


def _ln_mm(x, m_ref, g, b, eps=1e-6):
    mu = _mm(x, m_ref[...])
    xc = x - mu
    var = _mm(xc * xc, m_ref[...])
    return xc * lax.rsqrt(var + eps) * g + b


def _dense_body(g_ref, x_ref, xh_ref, fqv_ref, ph_ref,
                W1bd_ref, b1c_ref, W2bd_ref, b2c_ref,
                Wq_ref, bq_ref, Wk_ref, bk_ref, Wv_ref, bv_ref,
                Wc1_ref, bc1_ref, gc_ref, bcl_ref, Wc2_ref, bc2_ref,
                Wf1_ref, bf1_ref, gf_ref, bfl_ref, Wf2_ref, bf2_ref,
                Wm1bd_ref, bm1t_ref, gmt_ref, bmlt_ref, Wm2bd_ref, bm2t_ref,
                Wo_ref, bo_ref, atts_ref, ex_ref, m64_ref, m128_ref, out_ref):
    K, nb, _ = g_ref.shape
    H = Wc1_ref.shape[0]
    HH = Wq_ref.shape[1]
    NH = HH // H
    R = K * nb
    scale = 1.0 / (H ** 0.5)

    g = g_ref[...].reshape(R, g_ref.shape[2])
    cg = g[:, :H]
    pg = g[:, H:H + 3]
    xt = jnp.concatenate([x_ref[...]] * K, axis=0)
    inv = xt - pg

    proj = None
    for d in range(3):
        t = inv[:, d:d + 1] * fqv_ref[d:d + 1, :]
        proj = t if proj is None else proj + t
    hs = jnp.sin(proj + ph_ref[...])
    h1 = jax.nn.gelu(_mm(hs, W1bd_ref[...]) + b1c_ref[...])
    emb = _mm(h1, W2bd_ref[...]) + b2c_ref[...]
    embq = emb[:, :H]
    embv = emb[:, H:]

    q = _mm(embq, Wq_ref[...]) + bq_ref[...]
    k = _mm(cg, Wk_ref[...]) + bk_ref[...]
    v = _mm(cg, Wv_ref[...]) + bv_ref[...]

    xh = xh_ref[...]
    hc = jax.nn.gelu(_mm(xh, Wc1_ref[...]) + bc1_ref[...])
    hc = _ln_mm(hc, m64_ref, gc_ref[...], bcl_ref[...])
    gb = _mm(hc, Wc2_ref[...]) + bc2_ref[...]
    gamma_t = jnp.concatenate([gb[:, :H]] * K, axis=0)
    beta_t = jnp.concatenate([gb[:, H:]] * K, axis=0)
    embv = embv * (1.0 + gamma_t) + beta_t

    hf = jax.nn.gelu(_mm(embv, Wf1_ref[...]) + bf1_ref[...])
    hf = _ln_mm(hf, m64_ref, gf_ref[...], bfl_ref[...])
    vgb = _mm(hf, Wf2_ref[...]) + bf2_ref[...]
    v = v * (1.0 + vgb[:, :HH]) + vgb[:, HH:]

    prod = q * k
    att = _mm(prod, atts_ref[...]) * scale
    aks = [att[kk * nb:(kk + 1) * nb] for kk in range(K)]
    mx = aks[0]
    for kk in range(1, K):
        mx = jnp.maximum(mx, aks[kk])
    es = [jnp.exp(a - mx) for a in aks]
    s = es[0]
    for kk in range(1, K):
        s = s + es[kk]
    rin = 1.0 / s
    w_all = jnp.concatenate([e * rin for e in es], axis=0)
    w512 = _mm(w_all, ex_ref[...])

    vparts = []
    for hp in range(NH // 2):
        sl = slice(hp * 2 * H, (hp + 1) * 2 * H)
        vp = jax.nn.gelu(_mm(v[:, sl], Wm1bd_ref[...]) + bm1t_ref[...])
        vp = _ln_mm(vp, m128_ref, gmt_ref[...], bmlt_ref[...])
        vp = _mm(vp, Wm2bd_ref[...]) + bm2t_ref[...]
        vparts.append(vp)
    vf = jnp.concatenate(vparts, axis=1)

    yw = w512 * vf
    y = yw[:nb]
    for kk in range(1, K):
        y = y + yw[kk * nb:(kk + 1) * nb]
    out_ref[...] = _mm(y, Wo_ref[...]) + bo_ref[...]


def _sc_gather(table, idx_flat):
    M = idx_flat.shape[0]
    W = table.shape[1]
    NC, NS = 2, 16
    NW = NC * NS
    per_w = M // NW
    CH = 128
    n_ch = per_w // CH
    mesh = plsc.VectorSubcoreMesh(core_axis_name="c", subcore_axis_name="s")

    @functools.partial(
        pl.kernel, mesh=mesh,
        out_type=jax.ShapeDtypeStruct((M, W), jnp.float32),
        scratch_types=[
            pltpu.VMEM((CH,), jnp.int32),
            pltpu.VMEM((CH, W), jnp.float32),
            pltpu.SemaphoreType.DMA,
        ],
    )
    def gk(t_hbm, i_hbm, o_hbm, idx_v, rows_v, sem):
        wid = lax.axis_index("s") * NC + lax.axis_index("c")
        base_w = wid * per_w
        for j in range(n_ch):
            b = base_w + j * CH
            pltpu.sync_copy(i_hbm.at[pl.ds(b, CH)], idx_v)
            pltpu.async_copy(t_hbm.at[idx_v], rows_v, sem).wait()
            pltpu.sync_copy(rows_v, o_hbm.at[pl.ds(b, CH)])

    return gk(table, idx_flat)


def _blkdiag(a, b):
    z1 = jnp.zeros((a.shape[0], b.shape[1]), jnp.float32)
    z2 = jnp.zeros((b.shape[0], a.shape[1]), jnp.float32)
    return jnp.concatenate(
        [jnp.concatenate([a, z1], axis=1), jnp.concatenate([z2, b], axis=1)],
        axis=0)


def kernel(x, p, c, x_h, freqs_q, freqs_v, Wq1, bq1, Wq2, bq2, Wv1e, bv1e, Wv2e, bv2e, Wq, bq, Wk, bk, Wv, bv, Wc1, bc1, gc, bcl, Wc2, bc2, Wf1, bf1, gf, bfl, Wf2, bf2, Wm1, bm1, gm, bml, Wm2, bm2, Wo, bo):
    B, N, D = x.shape
    L = p.shape[1]
    H = c.shape[2]
    HH = Wq.shape[1]
    NH = HH // H
    BN = B * N

    pT = jnp.swapaxes(p, 1, 2)
    idx16 = pl.pallas_call(
        _topk_body,
        grid=(B, N // QB),
        in_specs=[
            pl.BlockSpec((1, QB, D), lambda b, i: (b, i, 0)),
            pl.BlockSpec((1, D, L), lambda b, i: (b, 0, 0)),
        ],
        out_specs=pl.BlockSpec((1, QB, IDX_PAD), lambda b, i: (b, i, 0)),
        out_shape=jax.ShapeDtypeStruct((B, N, IDX_PAD), jnp.int32),
    )(x, pT)
    idx9 = idx16[:, :, :TOP_K]
    offs = (jnp.arange(B, dtype=jnp.int32) * L)[:, None, None]
    idx_flat = (idx9 + offs).reshape(BN, TOP_K).T.reshape(-1)

    table = jnp.concatenate(
        [c.reshape(B * L, H), p.reshape(B * L, D),
         jnp.zeros((B * L, 128 - H - D), jnp.float32)], axis=1)
    G = _sc_gather(table, idx_flat)
    G3 = G.reshape(TOP_K, BN, 128)

    x_flat = x.reshape(BN, D)
    xh_flat = x_h.reshape(BN, H)
    row = lambda a: a.reshape(1, -1)
    full = lambda arr: pl.BlockSpec(arr.shape, lambda i: (0,) * arr.ndim)

    fq2 = (TWO_PI * FREQ_Q) * jnp.concatenate([freqs_q, freqs_q], axis=1)
    fv2 = (TWO_PI * FREQ_V) * jnp.concatenate([freqs_v, freqs_v], axis=1)
    fqv = jnp.concatenate([fq2, fv2], axis=1)
    nhalf = freqs_q.shape[1]
    ph1 = jnp.concatenate([jnp.zeros((1, nhalf), jnp.float32),
                           jnp.full((1, nhalf), 0.5 * PI, jnp.float32)], axis=1)
    ph2 = jnp.concatenate([ph1, ph1], axis=1)
    W1bd = _blkdiag(Wq1, Wv1e)
    b1c = row(jnp.concatenate([bq1, bv1e]))
    W2bd = _blkdiag(Wq2, Wv2e)
    b2c = row(jnp.concatenate([bq2, bv2e]))
    Wm1bd = _blkdiag(Wm1, Wm1)
    Wm2bd = _blkdiag(Wm2, Wm2)
    bm1t = row(jnp.concatenate([bm1, bm1]))
    bm2t = row(jnp.concatenate([bm2, bm2]))
    gmt = row(jnp.concatenate([gm, gm]))
    bmlt = row(jnp.concatenate([bml, bml]))
    atts = jnp.kron(jnp.eye(NH, dtype=jnp.float32),
                    jnp.ones((H, 1), jnp.float32))
    ex = jnp.kron(jnp.eye(NH, dtype=jnp.float32),
                  jnp.ones((1, H), jnp.float32))
    m64 = jnp.full((H, H), 1.0 / H, jnp.float32)
    m128 = _blkdiag(m64, m64)

    weights = [fqv, ph2,
               W1bd, b1c, W2bd, b2c,
               Wq, row(bq), Wk, row(bk), Wv, row(bv),
               Wc1, row(bc1), row(gc), row(bcl), Wc2, row(bc2),
               Wf1, row(bf1), row(gf), row(bfl), Wf2, row(bf2),
               Wm1bd, bm1t, gmt, bmlt, Wm2bd, bm2t,
               Wo, row(bo), atts, ex, m64, m128]
    out = pl.pallas_call(
        _dense_body,
        grid=(BN // NB,),
        in_specs=[
            pl.BlockSpec((TOP_K, NB, 128), lambda i: (0, i, 0)),
            pl.BlockSpec((NB, D), lambda i: (i, 0)),
            pl.BlockSpec((NB, H), lambda i: (i, 0)),
        ] + [full(a) for a in weights],
        out_specs=pl.BlockSpec((NB, H), lambda i: (i, 0)),
        out_shape=jax.ShapeDtypeStruct((BN, H), jnp.float32),
    )(G3, x_flat, xh_flat, *weights)
    return out.reshape(B, N, H)
